# baseline (speedup 1.0000x reference)
import functools

import jax
import jax.numpy as jnp
from jax import lax
from jax.experimental import pallas as pl
from jax.experimental.pallas import tpu as pltpu
from jax.experimental.pallas import tpu_sc as plsc

N = 10000
E = 320000
D = 128
H = 128
G = 64

NC = 2
NS = 16
NW = NC * NS
C = 64
EF = E + N
_NBUF = 3
_NIDX = 6
NCH = _NIDX * (-(-EF // (NW * C * _NIDX)))
EPW = NCH * C
EPAD = NW * EPW
NP = 10240
NA = 10016
RB = 2000
NB = N // RB


def _tc_pre_body(x_ref, w_ref, as_ref, ad_ref, h_ref, als_ref, ald_ref):
  h = jnp.dot(x_ref[...], w_ref[...], preferred_element_type=jnp.float32)
  h_ref[...] = h
  als_ref[...] = jnp.sum(h * as_ref[...], axis=1)[:, None]
  ald_ref[...] = jnp.sum(h * ad_ref[...], axis=1)[:, None]


def _tc_pre(x, W, a_src, a_dst):
  return pl.pallas_call(
      _tc_pre_body,
      grid=(NB,),
      in_specs=[
          pl.BlockSpec((RB, D), lambda i: (i, 0)),
          pl.BlockSpec((D, H), lambda i: (0, 0)),
          pl.BlockSpec((1, H), lambda i: (0, 0)),
          pl.BlockSpec((1, H), lambda i: (0, 0)),
      ],
      out_specs=[
          pl.BlockSpec((RB, H), lambda i: (i, 0)),
          pl.BlockSpec((RB, 1), lambda i: (i, 0)),
          pl.BlockSpec((RB, 1), lambda i: (i, 0)),
      ],
      out_shape=[
          jax.ShapeDtypeStruct((N, H), jnp.float32),
          jax.ShapeDtypeStruct((N, 1), jnp.float32),
          jax.ShapeDtypeStruct((N, 1), jnp.float32),
      ],
  )(x, W, a_src.reshape(1, H), a_dst.reshape(1, H))


def _tc_mid_body(acc0_ref, acc1_ref, den0_ref, den1_ref, b_ref, w_ref,
                 as_ref, ad_ref, h_ref, als_ref, ald_ref):
  num = acc0_ref[0] + acc1_ref[0]
  den = den0_ref[0] + den1_ref[0]
  h1 = num / (den + 1e-16) + b_ref[...]
  h1 = jnp.maximum(h1, 0.0)
  h = jnp.dot(h1, w_ref[...], preferred_element_type=jnp.float32)
  h_ref[...] = h
  als_ref[...] = jnp.sum(h * as_ref[...], axis=1)[:, None]
  ald_ref[...] = jnp.sum(h * ad_ref[...], axis=1)[:, None]


def _tc_mid(acc, den, b, W, a_src, a_dst):
  return pl.pallas_call(
      _tc_mid_body,
      grid=(NB,),
      in_specs=[
          pl.BlockSpec((1, RB, H), lambda i: (0, i, 0)),
          pl.BlockSpec((1, RB, H), lambda i: (1, i, 0)),
          pl.BlockSpec((1, RB, 1), lambda i: (0, i, 0)),
          pl.BlockSpec((1, RB, 1), lambda i: (1, i, 0)),
          pl.BlockSpec((1, H), lambda i: (0, 0)),
          pl.BlockSpec((D, H), lambda i: (0, 0)),
          pl.BlockSpec((1, H), lambda i: (0, 0)),
          pl.BlockSpec((1, H), lambda i: (0, 0)),
      ],
      out_specs=[
          pl.BlockSpec((RB, H), lambda i: (i, 0)),
          pl.BlockSpec((RB, 1), lambda i: (i, 0)),
          pl.BlockSpec((RB, 1), lambda i: (i, 0)),
      ],
      out_shape=[
          jax.ShapeDtypeStruct((N, H), jnp.float32),
          jax.ShapeDtypeStruct((N, 1), jnp.float32),
          jax.ShapeDtypeStruct((N, 1), jnp.float32),
      ],
  )(acc, acc, den, den, b.reshape(1, H), W,
    a_src.reshape(1, H), a_dst.reshape(1, H))


def _tc_post_body(acc0_ref, acc1_ref, den0_ref, den1_ref, b_ref, batch_ref,
                  out_ref, cnt_ref):
  i = pl.program_id(0)

  @pl.when(i == 0)
  def _init():
    out_ref[...] = jnp.zeros_like(out_ref)
    cnt_ref[...] = jnp.zeros_like(cnt_ref)

  num = acc0_ref[0] + acc1_ref[0]
  den = den0_ref[0] + den1_ref[0]
  h = num / (den + 1e-16) + b_ref[...]
  gids = lax.broadcasted_iota(jnp.int32, (RB, G), 1)
  onehot = (batch_ref[...] == gids).astype(jnp.float32)
  out_ref[...] += jnp.dot(onehot.T, h, preferred_element_type=jnp.float32)
  cnt_ref[...] += jnp.broadcast_to(jnp.sum(onehot, axis=0)[:, None], (G, H))

  @pl.when(i == NB - 1)
  def _fin():
    out_ref[...] = out_ref[...] / jnp.maximum(cnt_ref[...], 1.0)


def _tc_post(acc, den, b, batch):
  return pl.pallas_call(
      _tc_post_body,
      grid=(NB,),
      in_specs=[
          pl.BlockSpec((1, RB, H), lambda i: (0, i, 0)),
          pl.BlockSpec((1, RB, H), lambda i: (1, i, 0)),
          pl.BlockSpec((1, RB, 1), lambda i: (0, i, 0)),
          pl.BlockSpec((1, RB, 1), lambda i: (1, i, 0)),
          pl.BlockSpec((1, H), lambda i: (0, 0)),
          pl.BlockSpec((RB, 1), lambda i: (i, 0)),
      ],
      out_specs=pl.BlockSpec((G, H), lambda i: (0, 0)),
      out_shape=jax.ShapeDtypeStruct((G, H), jnp.float32),
      scratch_shapes=[pltpu.VMEM((G, H), jnp.float32)],
  )(acc, acc, den, den, b.reshape(1, H), batch.reshape(N, 1))


_RPS = NP // NS


def _sc_edge_body(h_hbm, als_hbm, ald_hbm, src_hbm, dst_hbm,
                  acc_hbm, den_hbm,
                  src_r, dst_r, rows_v, w_r, als_v, ald_v,
                  acc_sh, den_sh, gsem, ssem, isem):
  cid = lax.axis_index("c")
  sid = lax.axis_index("s")
  wid = sid * NC + cid
  ebase = pl.multiple_of(wid * EPW, EPW)

  pltpu.sync_copy(als_hbm, als_v)
  pltpu.sync_copy(ald_hbm, ald_v)

  @pl.loop(0, C)
  def _zrow(r):
    for j in range(H // 16):
      rows_v[0, r, pl.ds(j * 16, 16)] = jnp.zeros((16,), jnp.float32)

  for t in range(_RPS // C):
    base = sid * _RPS + t * C
    pltpu.sync_copy(rows_v.at[0], acc_sh.at[pl.ds(base, C)])
    pltpu.sync_copy(rows_v.at[0, 0, pl.ds(0, C)], den_sh.at[pl.ds(base, C)])
  plsc.subcore_barrier()

  def _eoff(k):
    return pl.ds(pl.multiple_of(wid * EPW + k * C, C), C)

  def _fetch_idx(k, q):
    pltpu.async_copy(src_hbm.at[_eoff(k)], src_r.at[pl.ds(q * C, C)],
                     isem.at[q])
    pltpu.async_copy(dst_hbm.at[wid, k], dst_r.at[q], isem.at[q])

  def _wait_idx(k, q):
    pltpu.make_async_copy(src_hbm.at[_eoff(k)], src_r.at[pl.ds(q * C, C)],
                          isem.at[q]).wait()
    pltpu.make_async_copy(dst_hbm.at[wid, k], dst_r.at[q], isem.at[q]).wait()

  def _gidx(k, q):
    return src_r.at[pl.ds(q * C, C)]

  def _start_gather(k, p, q):
    pltpu.async_copy(h_hbm.at[_gidx(k, q)], rows_v.at[p], gsem.at[p])

  def _wait_gather(k, p, q):
    pltpu.make_async_copy(h_hbm.at[_gidx(k, q)], rows_v.at[p],
                          gsem.at[p]).wait()

  def _start_scatter(k, p, q):
    pltpu.async_copy(rows_v.at[p], acc_sh.at[dst_r.at[q, 0]], ssem.at[p],
                     add=True)
    pltpu.async_copy(w_r.at[pl.ds(p * C, C)], den_sh.at[dst_r.at[q, 0]],
                     ssem.at[p], add=True)

  def _wait_scatter(k, p, q):
    pltpu.make_async_copy(
        rows_v.at[p], acc_sh.at[dst_r.at[q, 0]], ssem.at[p]).wait()
    pltpu.make_async_copy(
        w_r.at[pl.ds(p * C, C)], den_sh.at[dst_r.at[q, 0]], ssem.at[p]).wait()

  for k in range(4):
    _fetch_idx(k, k)
  _wait_idx(0, 0)
  _start_gather(0, 0, 0)
  _wait_idx(1, 1)
  _start_gather(1, 1, 1)

  @pl.loop(0, NCH, step=_NIDX)
  def _iter(k0):
    for ph in range(_NIDX):
      k = k0 + ph
      p = ph % _NBUF
      q = ph

      for g in range(C // 16):
        s16 = src_r[pl.ds(q * C + g * 16, 16)]
        d16 = dst_r[q, 0, pl.ds(g * 16, 16)]
        e = plsc.load_gather(als_v, [s16]) + plsc.load_gather(ald_v, [d16])
        e = jnp.where(e >= 0.0, e, 0.2 * e)
        w_r[pl.ds(p * C + g * 16, 16)] = jnp.exp(e)

      _wait_gather(k, p, q)

      @plsc.parallel_loop(0, C, unroll=8)
      def _scale(r):
        wv = plsc.load_gather(w_r, [lax.broadcast(p * C + r, (16,))])
        for j in range(H // 16):
          rows_v[p, r, pl.ds(j * 16, 16)] = (
              rows_v[p, r, pl.ds(j * 16, 16)] * wv)

      _start_scatter(k, p, q)

      @pl.when(k >= 1)
      def _drain():
        _wait_scatter(k - 1, (p + 2) % _NBUF, (q + 5) % _NIDX)

      @pl.when(k + 4 < NCH)
      def _f():
        _fetch_idx(k + 4, (ph + 4) % _NIDX)

      @pl.when(k + 2 < NCH)
      def _g():
        _wait_idx(k + 2, (ph + 2) % _NIDX)
        _start_gather(k + 2, (p + 2) % _NBUF, (ph + 2) % _NIDX)

  _wait_scatter(NCH - 1, (NCH - 1) % _NBUF, (NCH - 1) % _NIDX)
  plsc.subcore_barrier()

  base = sid * _RPS
  pltpu.sync_copy(acc_sh.at[pl.ds(base, _RPS)],
                  acc_hbm.at[cid, pl.ds(base, _RPS)])
  pltpu.sync_copy(den_sh.at[pl.ds(base, _RPS)],
                  den_hbm.at[cid, pl.ds(base, _RPS)])


@functools.lru_cache(maxsize=1)
def _get_sc_edge():
  return pl.kernel(
    _sc_edge_body,
    out_type=[
        jax.ShapeDtypeStruct((NC, NP, H), jnp.float32),
        jax.ShapeDtypeStruct((NC, NP), jnp.float32),
    ],
    mesh=plsc.VectorSubcoreMesh(core_axis_name="c", subcore_axis_name="s",
                                num_cores=NC, num_subcores=NS),
    compiler_params=pltpu.CompilerParams(needs_layout_passes=False),
    scratch_types=[
        pltpu.VMEM((_NIDX * C,), jnp.int32),
        pltpu.VMEM((_NIDX, 1, C), jnp.int32),
        pltpu.VMEM((_NBUF, C, H), jnp.float32),
        pltpu.VMEM((_NBUF * C,), jnp.float32),
        pltpu.VMEM((NA,), jnp.float32),
        pltpu.VMEM((NA,), jnp.float32),
        pltpu.VMEM_SHARED((NP, H), jnp.float32),
        pltpu.VMEM_SHARED((NP,), jnp.float32),
        pltpu.SemaphoreType.DMA((_NBUF,)),
        pltpu.SemaphoreType.DMA((_NBUF,)),
        pltpu.SemaphoreType.DMA((_NIDX,)),
    ],
  )


def kernel(x, edge_index, batch, W1, a_src1, a_dst1, b1,
           W2, a_src2, a_dst2, b2):
  src = edge_index[0]
  dst = edge_index[1]
  loop = jnp.arange(N, dtype=jnp.int32)
  pad = EPAD - EF
  srcf = jnp.concatenate([src, loop, jnp.zeros((pad,), jnp.int32)])
  dstf = jnp.concatenate([dst, loop, jnp.full((pad,), N, jnp.int32)])
  dstf = dstf.reshape(NW, NCH, 1, C)

  sc_edge = _get_sc_edge()
  zpad = jnp.zeros((NA - N,), jnp.float32)

  def _pad(a):
    return jnp.concatenate([a.reshape(N), zpad])

  h1, als1, ald1 = _tc_pre(x, W1, a_src1, a_dst1)
  acc1, den1 = sc_edge(h1, _pad(als1), _pad(ald1), srcf, dstf)
  den1r = den1[:, :N, None]
  h2, als2, ald2 = _tc_mid(acc1, den1r, b1, W2, a_src2, a_dst2)
  acc2, den2 = sc_edge(h2, _pad(als2), _pad(ald2), srcf, dstf)
  den2r = den2[:, :N, None]
  return _tc_post(acc2, den2r, b2, batch)


# speedup vs baseline: 1.0118x; 1.0118x over previous
import functools

import jax
import jax.numpy as jnp
from jax import lax
from jax.experimental import pallas as pl
from jax.experimental.pallas import tpu as pltpu
from jax.experimental.pallas import tpu_sc as plsc

N = 10000
E = 320000
D = 128
H = 128
G = 64

NC = 2
NS = 16
NW = NC * NS
C = 64
EF = E + N
_NBUF = 3
_NIDX = 6
NCH = _NIDX * (-(-EF // (NW * C * _NIDX)))
EPW = NCH * C
EPAD = NW * EPW
NP = 10240
NA = 10016
RB = 2000
NB = N // RB


def _tc_pre_body(x_ref, w_ref, as_ref, ad_ref, h_ref, als_ref, ald_ref):
  h = jnp.dot(x_ref[...], w_ref[...], preferred_element_type=jnp.float32)
  h_ref[...] = h
  als_ref[...] = jnp.sum(h * as_ref[...], axis=1)[:, None]
  ald_ref[...] = jnp.sum(h * ad_ref[...], axis=1)[:, None]


def _tc_pre(x, W, a_src, a_dst):
  return pl.pallas_call(
      _tc_pre_body,
      grid=(NB,),
      in_specs=[
          pl.BlockSpec((RB, D), lambda i: (i, 0)),
          pl.BlockSpec((D, H), lambda i: (0, 0)),
          pl.BlockSpec((1, H), lambda i: (0, 0)),
          pl.BlockSpec((1, H), lambda i: (0, 0)),
      ],
      out_specs=[
          pl.BlockSpec((RB, H), lambda i: (i, 0)),
          pl.BlockSpec((RB, 1), lambda i: (i, 0)),
          pl.BlockSpec((RB, 1), lambda i: (i, 0)),
      ],
      out_shape=[
          jax.ShapeDtypeStruct((N, H), jnp.float32),
          jax.ShapeDtypeStruct((N, 1), jnp.float32),
          jax.ShapeDtypeStruct((N, 1), jnp.float32),
      ],
  )(x, W, a_src.reshape(1, H), a_dst.reshape(1, H))


def _tc_mid_body(acc0_ref, acc1_ref, den0_ref, den1_ref, b_ref, w_ref,
                 as_ref, ad_ref, h_ref, als_ref, ald_ref):
  num = acc0_ref[0] + acc1_ref[0]
  den = den0_ref[0] + den1_ref[0]
  h1 = num / (den + 1e-16) + b_ref[...]
  h1 = jnp.maximum(h1, 0.0)
  h = jnp.dot(h1, w_ref[...], preferred_element_type=jnp.float32)
  h_ref[...] = h
  als_ref[...] = jnp.sum(h * as_ref[...], axis=1)[:, None]
  ald_ref[...] = jnp.sum(h * ad_ref[...], axis=1)[:, None]


def _tc_mid(acc, den, b, W, a_src, a_dst):
  return pl.pallas_call(
      _tc_mid_body,
      grid=(NB,),
      in_specs=[
          pl.BlockSpec((1, RB, H), lambda i: (0, i, 0)),
          pl.BlockSpec((1, RB, H), lambda i: (1, i, 0)),
          pl.BlockSpec((1, RB, 1), lambda i: (0, i, 0)),
          pl.BlockSpec((1, RB, 1), lambda i: (1, i, 0)),
          pl.BlockSpec((1, H), lambda i: (0, 0)),
          pl.BlockSpec((D, H), lambda i: (0, 0)),
          pl.BlockSpec((1, H), lambda i: (0, 0)),
          pl.BlockSpec((1, H), lambda i: (0, 0)),
      ],
      out_specs=[
          pl.BlockSpec((RB, H), lambda i: (i, 0)),
          pl.BlockSpec((RB, 1), lambda i: (i, 0)),
          pl.BlockSpec((RB, 1), lambda i: (i, 0)),
      ],
      out_shape=[
          jax.ShapeDtypeStruct((N, H), jnp.float32),
          jax.ShapeDtypeStruct((N, 1), jnp.float32),
          jax.ShapeDtypeStruct((N, 1), jnp.float32),
      ],
  )(acc, acc, den, den, b.reshape(1, H), W,
    a_src.reshape(1, H), a_dst.reshape(1, H))


def _tc_post_body(acc0_ref, acc1_ref, den0_ref, den1_ref, b_ref, batch_ref,
                  out_ref, cnt_ref):
  i = pl.program_id(0)

  @pl.when(i == 0)
  def _init():
    out_ref[...] = jnp.zeros_like(out_ref)
    cnt_ref[...] = jnp.zeros_like(cnt_ref)

  num = acc0_ref[0] + acc1_ref[0]
  den = den0_ref[0] + den1_ref[0]
  h = num / (den + 1e-16) + b_ref[...]
  gids = lax.broadcasted_iota(jnp.int32, (RB, G), 1)
  onehot = (batch_ref[...] == gids).astype(jnp.float32)
  out_ref[...] += jnp.dot(onehot.T, h, preferred_element_type=jnp.float32)
  cnt_ref[...] += jnp.broadcast_to(jnp.sum(onehot, axis=0)[:, None], (G, H))

  @pl.when(i == NB - 1)
  def _fin():
    out_ref[...] = out_ref[...] / jnp.maximum(cnt_ref[...], 1.0)


def _tc_post(acc, den, b, batch):
  return pl.pallas_call(
      _tc_post_body,
      grid=(NB,),
      in_specs=[
          pl.BlockSpec((1, RB, H), lambda i: (0, i, 0)),
          pl.BlockSpec((1, RB, H), lambda i: (1, i, 0)),
          pl.BlockSpec((1, RB, 1), lambda i: (0, i, 0)),
          pl.BlockSpec((1, RB, 1), lambda i: (1, i, 0)),
          pl.BlockSpec((1, H), lambda i: (0, 0)),
          pl.BlockSpec((RB, 1), lambda i: (i, 0)),
      ],
      out_specs=pl.BlockSpec((G, H), lambda i: (0, 0)),
      out_shape=jax.ShapeDtypeStruct((G, H), jnp.float32),
      scratch_shapes=[pltpu.VMEM((G, H), jnp.float32)],
  )(acc, acc, den, den, b.reshape(1, H), batch.reshape(N, 1))


_RPS = NP // NS


def _sc_edge_body(h_hbm, als_hbm, ald_hbm, src_hbm, dst_hbm,
                  acc_hbm, den_hbm,
                  src_r, dst_r, rows_v, w_r, als_v, ald_v,
                  acc_sh, den_sh, gsem, ssem, isem):
  cid = lax.axis_index("c")
  sid = lax.axis_index("s")
  wid = sid * NC + cid
  ebase = pl.multiple_of(wid * EPW, EPW)

  pltpu.sync_copy(als_hbm, als_v)
  pltpu.sync_copy(ald_hbm, ald_v)

  @pl.loop(0, C)
  def _zrow(r):
    for j in range(H // 16):
      rows_v[0, r, pl.ds(j * 16, 16)] = jnp.zeros((16,), jnp.float32)

  for t in range(_RPS // C):
    base = sid * _RPS + t * C
    pltpu.sync_copy(rows_v.at[0], acc_sh.at[pl.ds(base, C)])
    pltpu.sync_copy(rows_v.at[0, 0, pl.ds(0, C)], den_sh.at[pl.ds(base, C)])
  plsc.subcore_barrier()

  def _eoff(k):
    return pl.ds(pl.multiple_of(wid * EPW + k * C, C), C)

  def _fetch_idx(k, q):
    pltpu.async_copy(src_hbm.at[_eoff(k)], src_r.at[pl.ds(q * C, C)],
                     isem.at[q])
    pltpu.async_copy(dst_hbm.at[wid, k], dst_r.at[q], isem.at[q])

  def _wait_idx(k, q):
    pltpu.make_async_copy(src_hbm.at[_eoff(k)], src_r.at[pl.ds(q * C, C)],
                          isem.at[q]).wait()
    pltpu.make_async_copy(dst_hbm.at[wid, k], dst_r.at[q], isem.at[q]).wait()

  def _gidx(k, q):
    return src_r.at[pl.ds(q * C, C)]

  def _start_gather(k, p, q):
    pltpu.async_copy(h_hbm.at[_gidx(k, q)], rows_v.at[p], gsem.at[p])

  def _wait_gather(k, p, q):
    pltpu.make_async_copy(h_hbm.at[_gidx(k, q)], rows_v.at[p],
                          gsem.at[p]).wait()

  def _start_scatter(k, p, q):
    pltpu.async_copy(rows_v.at[p], acc_sh.at[dst_r.at[q, 0]], ssem.at[p],
                     add=True)
    pltpu.async_copy(w_r.at[pl.ds(p * C, C)], den_sh.at[dst_r.at[q, 0]],
                     ssem.at[p], add=True)

  def _wait_scatter(k, p, q):
    pltpu.make_async_copy(
        rows_v.at[p], acc_sh.at[dst_r.at[q, 0]], ssem.at[p]).wait()
    pltpu.make_async_copy(
        w_r.at[pl.ds(p * C, C)], den_sh.at[dst_r.at[q, 0]], ssem.at[p]).wait()

  for k in range(4):
    _fetch_idx(k, k)
  _wait_idx(0, 0)
  _start_gather(0, 0, 0)
  _wait_idx(1, 1)
  _start_gather(1, 1, 1)

  @pl.loop(0, NCH, step=_NIDX)
  def _iter(k0):
    for ph in range(_NIDX):
      k = k0 + ph
      p = ph % _NBUF
      q = ph

      for g in range(C // 16):
        s16 = src_r[pl.ds(q * C + g * 16, 16)]
        d16 = dst_r[q, 0, pl.ds(g * 16, 16)]
        e = plsc.load_gather(als_v, [s16]) + plsc.load_gather(ald_v, [d16])
        e = jnp.where(e >= 0.0, e, 0.2 * e)
        w_r[pl.ds(p * C + g * 16, 16)] = jnp.exp(e)

      _wait_gather(k, p, q)

      @plsc.parallel_loop(0, C, unroll=4)
      def _scale(r):
        wv = plsc.load_gather(w_r, [lax.broadcast(p * C + r, (16,))])
        for j in range(H // 16):
          rows_v[p, r, pl.ds(j * 16, 16)] = (
              rows_v[p, r, pl.ds(j * 16, 16)] * wv)

      _start_scatter(k, p, q)

      @pl.when(k >= 1)
      def _drain():
        _wait_scatter(k - 1, (p + 2) % _NBUF, (q + 5) % _NIDX)

      @pl.when(k + 4 < NCH)
      def _f():
        _fetch_idx(k + 4, (ph + 4) % _NIDX)

      @pl.when(k + 2 < NCH)
      def _g():
        _wait_idx(k + 2, (ph + 2) % _NIDX)
        _start_gather(k + 2, (p + 2) % _NBUF, (ph + 2) % _NIDX)

  _wait_scatter(NCH - 1, (NCH - 1) % _NBUF, (NCH - 1) % _NIDX)
  plsc.subcore_barrier()

  base = sid * _RPS
  pltpu.sync_copy(acc_sh.at[pl.ds(base, _RPS)],
                  acc_hbm.at[cid, pl.ds(base, _RPS)])
  pltpu.sync_copy(den_sh.at[pl.ds(base, _RPS)],
                  den_hbm.at[cid, pl.ds(base, _RPS)])


@functools.lru_cache(maxsize=1)
def _get_sc_edge():
  return pl.kernel(
    _sc_edge_body,
    out_type=[
        jax.ShapeDtypeStruct((NC, NP, H), jnp.float32),
        jax.ShapeDtypeStruct((NC, NP), jnp.float32),
    ],
    mesh=plsc.VectorSubcoreMesh(core_axis_name="c", subcore_axis_name="s",
                                num_cores=NC, num_subcores=NS),
    compiler_params=pltpu.CompilerParams(needs_layout_passes=False),
    scratch_types=[
        pltpu.VMEM((_NIDX * C,), jnp.int32),
        pltpu.VMEM((_NIDX, 1, C), jnp.int32),
        pltpu.VMEM((_NBUF, C, H), jnp.float32),
        pltpu.VMEM((_NBUF * C,), jnp.float32),
        pltpu.VMEM((NA,), jnp.float32),
        pltpu.VMEM((NA,), jnp.float32),
        pltpu.VMEM_SHARED((NP, H), jnp.float32),
        pltpu.VMEM_SHARED((NP,), jnp.float32),
        pltpu.SemaphoreType.DMA((_NBUF,)),
        pltpu.SemaphoreType.DMA((_NBUF,)),
        pltpu.SemaphoreType.DMA((_NIDX,)),
    ],
  )


def kernel(x, edge_index, batch, W1, a_src1, a_dst1, b1,
           W2, a_src2, a_dst2, b2):
  src = edge_index[0]
  dst = edge_index[1]
  loop = jnp.arange(N, dtype=jnp.int32)
  pad = EPAD - EF
  srcf = jnp.concatenate([src, loop, jnp.zeros((pad,), jnp.int32)])
  dstf = jnp.concatenate([dst, loop, jnp.full((pad,), N, jnp.int32)])
  dstf = dstf.reshape(NW, NCH, 1, C)

  sc_edge = _get_sc_edge()
  zpad = jnp.zeros((NA - N,), jnp.float32)

  def _pad(a):
    return jnp.concatenate([a.reshape(N), zpad])

  h1, als1, ald1 = _tc_pre(x, W1, a_src1, a_dst1)
  acc1, den1 = sc_edge(h1, _pad(als1), _pad(ald1), srcf, dstf)
  den1r = den1[:, :N, None]
  h2, als2, ald2 = _tc_mid(acc1, den1r, b1, W2, a_src2, a_dst2)
  acc2, den2 = sc_edge(h2, _pad(als2), _pad(ald2), srcf, dstf)
  den2r = den2[:, :N, None]
  return _tc_post(acc2, den2r, b2, batch)
